# Initial kernel scaffold; baseline (speedup 1.0000x reference)
import jax
import jax.numpy as jnp
from jax.experimental import pallas as pl


def kernel(bin_ids, subbin_ids, bin_tables, subbin_tables):
    raise NotImplementedError("write your pallas kernel here")


# speedup vs baseline: 6.7779x; 6.7779x over previous
import functools

import jax
import jax.numpy as jnp
from jax import lax
from jax.experimental import pallas as pl
from jax.experimental.pallas import tpu as pltpu
from jax.experimental.pallas import tpu_sc as plsc

MAX_LEN = 1000
NCOL = 26
D = 64
B = 16384

NC = 2
NS = 16
NW = NC * NS
L = 16

C = 416
NROWS = B * NCOL
NCHUNK = NROWS // C
IPT = NCHUNK // NW
GSUB = 4
GC = C // GSUB


def _mk_kernel():
    mesh = plsc.VectorSubcoreMesh(core_axis_name="c", subcore_axis_name="s")

    @functools.partial(
        pl.kernel,
        mesh=mesh,
        compiler_params=pltpu.CompilerParams(use_tc_tiling_on_sc=False),
        out_type=jax.ShapeDtypeStruct((NROWS, D), jnp.float32),
        scratch_types=[
            pltpu.VMEM((C,), jnp.int32),
            pltpu.VMEM((C,), jnp.int32),
            pltpu.VMEM((C,), jnp.int32),
            pltpu.VMEM((C, D), jnp.float32),
            pltpu.VMEM((C, D), jnp.float32),
            pltpu.SemaphoreType.DMA,
        ],
    )
    def k(ids_b_h, ids_s_h, bin_t_h, sub_t_h, pat_h, out_h,
          idx_b, idx_s, pat_v, rows_a, rows_b, sem):
        wid = lax.axis_index("s") * NC + lax.axis_index("c")
        pltpu.sync_copy(pat_h, pat_v)

        def item_body(t, _):
            g = wid * IPT + t
            r0 = g * C

            cpb = pltpu.async_copy(ids_b_h.at[pl.ds(r0, C)], idx_b, sem)
            cps = pltpu.async_copy(ids_s_h.at[pl.ds(r0, C)], idx_s, sem)
            cpb.wait()
            cps.wait()

            for kk in range(C // L):
                sl = pl.ds(kk * L, L)
                p = pat_v[sl]
                idx_b[sl] = idx_b[sl] + p
                idx_s[sl] = idx_s[sl] + p

            handles = []
            for j in range(GSUB):
                isl = pl.ds(j * GC, GC)
                handles.append(pltpu.async_copy(
                    bin_t_h.at[idx_b.at[isl]], rows_a.at[isl], sem))
                handles.append(pltpu.async_copy(
                    sub_t_h.at[idx_s.at[isl]], rows_b.at[isl], sem))
            for h in handles:
                h.wait()

            def add_body(r, _):
                row = r * 4
                for rr in range(4):
                    for k2 in range(D // L):
                        sl2 = pl.ds(k2 * L, L)
                        rows_a[row + rr, sl2] = (
                            rows_a[row + rr, sl2] + rows_b[row + rr, sl2])
                return _

            lax.fori_loop(0, C // 4, add_body, None)

            pltpu.sync_copy(rows_a, out_h.at[pl.ds(r0, C)])
            return _

        lax.fori_loop(0, IPT, item_body, None)

    return k


_sc_kernel = _mk_kernel()


def kernel(bin_ids, subbin_ids, bin_tables, subbin_tables):
    ids_b = bin_ids.astype(jnp.int32).reshape(-1)
    ids_s = subbin_ids.astype(jnp.int32).reshape(-1)
    bin_t = bin_tables.reshape(NCOL * MAX_LEN, D)
    sub_t = subbin_tables.reshape(NCOL * MAX_LEN, D)
    pat = jnp.tile(jnp.arange(NCOL, dtype=jnp.int32) * MAX_LEN, C // NCOL)
    out = _sc_kernel(ids_b, ids_s, bin_t, sub_t, pat)
    return out.reshape(B, NCOL, D)
